# Initial kernel scaffold; baseline (speedup 1.0000x reference)
import jax
import jax.numpy as jnp
from jax.experimental import pallas as pl


def kernel(x, codebook):
    raise NotImplementedError("write your pallas kernel here")


# speedup vs baseline: 7.4999x; 7.4999x over previous
import jax
import jax.numpy as jnp
from jax.experimental import pallas as pl

_N = 16384
_K = 8192
_D = 32
_BLK = 512
_HALF = _K // 2


def _bf16_rne(v):
    u = jax.lax.bitcast_convert_type(v, jnp.uint32)
    u = (u + jnp.uint32(0x7FFF) + ((u >> 16) & jnp.uint32(1))) & jnp.uint32(0xFFFF0000)
    return jax.lax.bitcast_convert_type(u, jnp.float32)


def _vq_body(xb16_ref, cb16_ref, l2_ref, c2_ref, q_ref, idx_ref):
    xb = xb16_ref[...]
    cb = cb16_ref[...]
    l2 = l2_ref[...]
    c2 = c2_ref[...]
    cl = jax.lax.dot_general(
        xb, cb, (((1,), (1,)), ((), ())),
        preferred_element_type=jnp.float32)
    d = (l2 - cl * 2.0) + c2

    iota = jax.lax.broadcasted_iota(jnp.int32, (_BLK, _HALF), 1)
    d0 = d[:, :_HALF]
    d1 = d[:, _HALF:]
    m0 = jnp.min(d0, axis=1, keepdims=True)
    m1 = jnp.min(d1, axis=1, keepdims=True)
    i0 = jnp.min(jnp.where(d0 == m0, iota, jnp.int32(_K)), axis=1)
    i1 = jnp.min(jnp.where(d1 == m1, iota, jnp.int32(_K)), axis=1) + jnp.int32(_HALF)
    win1 = (m1 < _bf16_rne(m0))[:, 0]
    idx = jnp.where(win1, i1, i0)

    iota_k = jax.lax.broadcasted_iota(jnp.int32, (_BLK, _K), 1)
    oh = jnp.where(iota_k == idx[:, None],
                   jnp.float32(1.0), jnp.float32(0.0)).astype(jnp.bfloat16)
    q = jax.lax.dot_general(
        oh, cb, (((1,), (0,)), ((), ())),
        preferred_element_type=jnp.float32)
    q_ref[...] = q
    idx_ref[...] = idx.reshape(1, 1, _BLK)


def kernel(x, codebook):
    nblk = _N // _BLK
    xb16 = x.astype(jnp.bfloat16)
    cb16 = codebook.astype(jnp.bfloat16)
    l2 = jnp.sum(x ** 2, axis=1, keepdims=True)
    c2 = jnp.sum(codebook ** 2, axis=1)[None, :]
    q, idx3 = pl.pallas_call(
        _vq_body,
        grid=(nblk,),
        in_specs=[
            pl.BlockSpec((_BLK, _D), lambda i: (i, 0)),
            pl.BlockSpec((_K, _D), lambda i: (0, 0)),
            pl.BlockSpec((_BLK, 1), lambda i: (i, 0)),
            pl.BlockSpec((1, _K), lambda i: (0, 0)),
        ],
        out_specs=[
            pl.BlockSpec((_BLK, _D), lambda i: (i, 0)),
            pl.BlockSpec((1, 1, _BLK), lambda i: (i, 0, 0)),
        ],
        out_shape=[
            jax.ShapeDtypeStruct((_N, _D), jnp.float32),
            jax.ShapeDtypeStruct((nblk, 1, _BLK), jnp.int32),
        ],
    )(xb16, cb16, l2, c2)
    return q, idx3.reshape(_N)


# speedup vs baseline: 12.9810x; 1.7308x over previous
import functools

import jax
import jax.numpy as jnp
from jax import lax
from jax.experimental import pallas as pl
from jax.experimental.pallas import tpu as pltpu
from jax.experimental.pallas import tpu_sc as plsc

_N = 16384
_K = 8192
_D = 32
_BLK = 512
_HALF = _K // 2

_NW = 32
_BPW = _N // _NW
_CHUNK = 128


def _bf16_rne(v):
    u = jax.lax.bitcast_convert_type(v, jnp.uint32)
    u = (u + jnp.uint32(0x7FFF) + ((u >> 16) & jnp.uint32(1))) & jnp.uint32(0xFFFF0000)
    return jax.lax.bitcast_convert_type(u, jnp.float32)


def _vq_body(xb16_ref, cb16_ref, l2_ref, c2_ref, idx_ref):
    xb = xb16_ref[...]
    cb = cb16_ref[...]
    l2 = l2_ref[...]
    c2 = c2_ref[...]
    cl = jax.lax.dot_general(
        xb, cb, (((1,), (1,)), ((), ())),
        preferred_element_type=jnp.float32)
    d = (l2 - cl * 2.0) + c2

    iota = jax.lax.broadcasted_iota(jnp.int32, (_BLK, _HALF), 1)
    d0 = d[:, :_HALF]
    d1 = d[:, _HALF:]
    m0 = jnp.min(d0, axis=1, keepdims=True)
    m1 = jnp.min(d1, axis=1, keepdims=True)
    i0 = jnp.min(jnp.where(d0 == m0, iota, jnp.int32(_K)), axis=1)
    i1 = jnp.min(jnp.where(d1 == m1, iota, jnp.int32(_K)), axis=1) + jnp.int32(_HALF)
    win1 = (m1 < _bf16_rne(m0))[:, 0]
    idx_ref[...] = jnp.where(win1, i1, i0).reshape(1, 1, _BLK)


def _closest(xb16, cb16, l2, c2):
    nblk = _N // _BLK
    idx3 = pl.pallas_call(
        _vq_body,
        grid=(nblk,),
        in_specs=[
            pl.BlockSpec((_BLK, _D), lambda i: (i, 0)),
            pl.BlockSpec((_K, _D), lambda i: (0, 0)),
            pl.BlockSpec((_BLK, 1), lambda i: (i, 0)),
            pl.BlockSpec((1, _K), lambda i: (0, 0)),
        ],
        out_specs=pl.BlockSpec((1, 1, _BLK), lambda i: (i, 0, 0)),
        out_shape=jax.ShapeDtypeStruct((nblk, 1, _BLK), jnp.int32),
    )(xb16, cb16, l2, c2)
    return idx3.reshape(_N)


def _gather_body(cbq_hbm, idx_hbm, out_hbm, idx_v, rows_v, sem):
    wid = lax.axis_index("s") * 2 + lax.axis_index("c")
    base = wid * _BPW
    pltpu.sync_copy(idx_hbm.at[wid], idx_v)
    copies = []
    for j in range(_BPW // _CHUNK):
        copies.append(pltpu.async_copy(
            cbq_hbm.at[idx_v.at[j]], rows_v.at[pl.ds(j * _CHUNK, _CHUNK)],
            sem))
    for c in copies:
        c.wait()
    pltpu.sync_copy(rows_v, out_hbm.at[pl.ds(base, _BPW)])


def _sc_gather(cbq, closest):
    k = pl.kernel(
        _gather_body,
        mesh=plsc.VectorSubcoreMesh(core_axis_name="c", subcore_axis_name="s"),
        out_type=jax.ShapeDtypeStruct((_N, _D), jnp.float32),
        scratch_types=[
            pltpu.VMEM((_BPW // _CHUNK, _CHUNK), jnp.int32),
            pltpu.VMEM((_BPW, _D), jnp.float32),
            pltpu.SemaphoreType.DMA,
        ],
        compiler_params=pltpu.CompilerParams(use_tc_tiling_on_sc=False),
    )
    return k(cbq, closest.reshape(_NW, _BPW // _CHUNK, _CHUNK))


def kernel(x, codebook):
    xb16 = x.astype(jnp.bfloat16)
    cb16 = codebook.astype(jnp.bfloat16)
    cbq = cb16.astype(jnp.float32)
    l2 = jnp.sum(x ** 2, axis=1, keepdims=True)
    c2 = jnp.sum(codebook ** 2, axis=1)[None, :]
    closest = _closest(xb16, cb16, l2, c2)
    quantized = _sc_gather(cbq, closest)
    return quantized, closest


# speedup vs baseline: 20.0512x; 1.5447x over previous
import functools

import jax
import jax.numpy as jnp
from jax import lax
from jax.experimental import pallas as pl
from jax.experimental.pallas import tpu as pltpu
from jax.experimental.pallas import tpu_sc as plsc

_N = 16384
_K = 8192
_D = 32
_BLK = 512
_HALF = _K // 2

_NW = 32
_BPW = _N // _NW
_CHUNK = 128


def _bf16_rne(v):
    u = jax.lax.bitcast_convert_type(v, jnp.uint32)
    u = (u + jnp.uint32(0x7FFF) + ((u >> 16) & jnp.uint32(1))) & jnp.uint32(0xFFFF0000)
    return jax.lax.bitcast_convert_type(u, jnp.float32)


_SB = 8


def _lex_argmin_window_t(clt, l2hr, c2t, lo, hi):
    sub = jax.lax.broadcasted_iota(jnp.int32, (_SB, _BLK), 0)
    acc_v = jnp.full((_SB, _BLK), jnp.inf, jnp.float32)
    acc_j = jnp.zeros((_SB, _BLK), jnp.int32)
    for j in range(lo, hi, _SB):
        c2j = jnp.broadcast_to(c2t[j:j + _SB, :], (_SB, _BLK))
        dj = (l2hr - clt[j:j + _SB, :]) + c2j
        mask = dj < acc_v
        acc_j = jnp.where(mask, jnp.int32(j), acc_j)
        acc_v = jnp.minimum(acc_v, dj)
    cand_i = acc_j + sub
    m_all = acc_v
    for s in (4, 2, 1):
        m_all = jnp.minimum(m_all, pltpu.roll(m_all, s, axis=0))
    cand = jnp.where(acc_v == m_all, cand_i, jnp.int32(2147483647))
    for s in (4, 2, 1):
        cand = jnp.minimum(cand, pltpu.roll(cand, s, axis=0))
    return m_all[0:1, :], cand[0:1, :]


def _vq_body(xb16_ref, cb16_ref, l2h_ref, c2t_ref, idx_ref):
    xb = xb16_ref[...]
    cb = cb16_ref[...]
    l2hr = l2h_ref[...]
    c2t = c2t_ref[...]
    clt = jax.lax.dot_general(
        cb, xb, (((1,), (1,)), ((), ())),
        preferred_element_type=jnp.float32)
    m0, i0 = _lex_argmin_window_t(clt, l2hr, c2t, 0, _HALF)
    m1, i1 = _lex_argmin_window_t(clt, l2hr, c2t, _HALF, _K)
    win1 = m1 < _bf16_rne(m0)
    idx_ref[...] = jnp.where(win1, i1, i0).reshape(1, 1, _BLK)


def _closest(xb16, cb16, l2h, c2t):
    nblk = _N // _BLK
    idx3 = pl.pallas_call(
        _vq_body,
        grid=(nblk,),
        in_specs=[
            pl.BlockSpec((_BLK, _D), lambda i: (i, 0)),
            pl.BlockSpec((_K, _D), lambda i: (0, 0)),
            pl.BlockSpec((1, _BLK), lambda i: (0, i)),
            pl.BlockSpec((_K, 1), lambda i: (0, 0)),
        ],
        out_specs=pl.BlockSpec((1, 1, _BLK), lambda i: (i, 0, 0)),
        out_shape=jax.ShapeDtypeStruct((nblk, 1, _BLK), jnp.int32),
    )(xb16, cb16, l2h, c2t)
    return idx3.reshape(_N)


def _gather_body(cbq_hbm, idx_hbm, out_hbm, idx_v, rows_v, sem):
    wid = lax.axis_index("s") * 2 + lax.axis_index("c")
    base = wid * _BPW
    pltpu.sync_copy(idx_hbm.at[wid], idx_v)
    copies = []
    for j in range(_BPW // _CHUNK):
        copies.append(pltpu.async_copy(
            cbq_hbm.at[idx_v.at[j]], rows_v.at[pl.ds(j * _CHUNK, _CHUNK)],
            sem))
    for c in copies:
        c.wait()
    pltpu.sync_copy(rows_v, out_hbm.at[pl.ds(base, _BPW)])


def _sc_gather(cbq, closest):
    k = pl.kernel(
        _gather_body,
        mesh=plsc.VectorSubcoreMesh(core_axis_name="c", subcore_axis_name="s"),
        out_type=jax.ShapeDtypeStruct((_N, _D), jnp.float32),
        scratch_types=[
            pltpu.VMEM((_BPW // _CHUNK, _CHUNK), jnp.int32),
            pltpu.VMEM((_BPW, _D), jnp.float32),
            pltpu.SemaphoreType.DMA,
        ],
        compiler_params=pltpu.CompilerParams(use_tc_tiling_on_sc=False),
    )
    return k(cbq, closest.reshape(_NW, _BPW // _CHUNK, _CHUNK))


def kernel(x, codebook):
    xb16 = x.astype(jnp.bfloat16)
    cb16 = codebook.astype(jnp.bfloat16)
    cbq = cb16.astype(jnp.float32)
    l2h = (jnp.sum(x ** 2, axis=1, keepdims=True) * 0.5).reshape(1, _N)
    c2t = (jnp.sum(codebook ** 2, axis=1)[None, :] * 0.5).reshape(_K, 1)
    closest = _closest(xb16, cb16, l2h, c2t)
    quantized = _sc_gather(cbq, closest)
    return quantized, closest


# speedup vs baseline: 20.6502x; 1.0299x over previous
import functools

import jax
import jax.numpy as jnp
from jax import lax
from jax.experimental import pallas as pl
from jax.experimental.pallas import tpu as pltpu
from jax.experimental.pallas import tpu_sc as plsc

_N = 16384
_K = 8192
_D = 32
_BLK = 1024
_HALF = _K // 2

_NW = 32
_BPW = _N // _NW
_CHUNK = 128


def _bf16_rne(v):
    u = jax.lax.bitcast_convert_type(v, jnp.uint32)
    u = (u + jnp.uint32(0x7FFF) + ((u >> 16) & jnp.uint32(1))) & jnp.uint32(0xFFFF0000)
    return jax.lax.bitcast_convert_type(u, jnp.float32)


_SB = 8


def _lex_argmin_window_t(clt, l2hr, c2t, lo, hi):
    sub = jax.lax.broadcasted_iota(jnp.int32, (_SB, _BLK), 0)
    acc_v = jnp.full((_SB, _BLK), jnp.inf, jnp.float32)
    acc_j = jnp.zeros((_SB, _BLK), jnp.int32)
    for j in range(lo, hi, _SB):
        c2j = jnp.broadcast_to(c2t[j:j + _SB, :], (_SB, _BLK))
        dj = (l2hr - clt[j:j + _SB, :]) + c2j
        mask = dj < acc_v
        acc_j = jnp.where(mask, jnp.int32(j), acc_j)
        acc_v = jnp.minimum(acc_v, dj)
    cand_i = acc_j + sub
    m_all = acc_v
    for s in (4, 2, 1):
        m_all = jnp.minimum(m_all, pltpu.roll(m_all, s, axis=0))
    cand = jnp.where(acc_v == m_all, cand_i, jnp.int32(2147483647))
    for s in (4, 2, 1):
        cand = jnp.minimum(cand, pltpu.roll(cand, s, axis=0))
    return m_all[0:1, :], cand[0:1, :]


def _vq_body(xb16_ref, cb16_ref, l2h_ref, c2t_ref, idx_ref):
    xb = xb16_ref[...]
    cb = cb16_ref[...]
    l2hr = l2h_ref[...]
    c2t = c2t_ref[...]
    clt = jax.lax.dot_general(
        cb, xb, (((1,), (1,)), ((), ())),
        preferred_element_type=jnp.float32)
    m0, i0 = _lex_argmin_window_t(clt, l2hr, c2t, 0, _HALF)
    m1, i1 = _lex_argmin_window_t(clt, l2hr, c2t, _HALF, _K)
    win1 = m1 < _bf16_rne(m0)
    idx_ref[...] = jnp.where(win1, i1, i0).reshape(1, 1, _BLK)


def _closest(xb16, cb16, l2h, c2t):
    nblk = _N // _BLK
    idx3 = pl.pallas_call(
        _vq_body,
        grid=(nblk,),
        in_specs=[
            pl.BlockSpec((_BLK, _D), lambda i: (i, 0)),
            pl.BlockSpec((_K, _D), lambda i: (0, 0)),
            pl.BlockSpec((1, _BLK), lambda i: (0, i)),
            pl.BlockSpec((_K, 1), lambda i: (0, 0)),
        ],
        out_specs=pl.BlockSpec((1, 1, _BLK), lambda i: (i, 0, 0)),
        out_shape=jax.ShapeDtypeStruct((nblk, 1, _BLK), jnp.int32),
    )(xb16, cb16, l2h, c2t)
    return idx3.reshape(_N)


def _gather_body(cbq_hbm, idx_hbm, out_hbm, idx_v, rows_v, sem):
    wid = lax.axis_index("s") * 2 + lax.axis_index("c")
    base = wid * _BPW
    pltpu.sync_copy(idx_hbm.at[wid], idx_v)
    copies = []
    for j in range(_BPW // _CHUNK):
        copies.append(pltpu.async_copy(
            cbq_hbm.at[idx_v.at[j]], rows_v.at[pl.ds(j * _CHUNK, _CHUNK)],
            sem))
    for c in copies:
        c.wait()
    pltpu.sync_copy(rows_v, out_hbm.at[pl.ds(base, _BPW)])


def _sc_gather(cbq, closest):
    k = pl.kernel(
        _gather_body,
        mesh=plsc.VectorSubcoreMesh(core_axis_name="c", subcore_axis_name="s"),
        out_type=jax.ShapeDtypeStruct((_N, _D), jnp.float32),
        scratch_types=[
            pltpu.VMEM((_BPW // _CHUNK, _CHUNK), jnp.int32),
            pltpu.VMEM((_BPW, _D), jnp.float32),
            pltpu.SemaphoreType.DMA,
        ],
        compiler_params=pltpu.CompilerParams(use_tc_tiling_on_sc=False),
    )
    return k(cbq, closest.reshape(_NW, _BPW // _CHUNK, _CHUNK))


def kernel(x, codebook):
    xb16 = x.astype(jnp.bfloat16)
    cb16 = codebook.astype(jnp.bfloat16)
    cbq = cb16.astype(jnp.float32)
    l2h = (jnp.sum(x ** 2, axis=1, keepdims=True) * 0.5).reshape(1, _N)
    c2t = (jnp.sum(codebook ** 2, axis=1)[None, :] * 0.5).reshape(_K, 1)
    closest = _closest(xb16, cb16, l2h, c2t)
    quantized = _sc_gather(cbq, closest)
    return quantized, closest
